# Initial kernel scaffold; baseline (speedup 1.0000x reference)
import jax
import jax.numpy as jnp
from jax.experimental import pallas as pl


def kernel(tid, node_emb_in, node_emb_out, tid_table, adp_emb):
    raise NotImplementedError("write your pallas kernel here")


# speedup vs baseline: 3.7967x; 3.7967x over previous
import functools

import jax
import jax.numpy as jnp
from jax import lax
from jax.experimental import pallas as pl
from jax.experimental.pallas import tpu as pltpu
from jax.experimental.pallas import tpu_sc as plsc

TID_DIM = 32
N_TOTAL = 64 * 12 * 5000
NW = 32
PER_W = N_TOTAL // NW
CHUNK = 96
NCHUNK = PER_W // CHUNK

_mesh = plsc.VectorSubcoreMesh(core_axis_name="c", subcore_axis_name="s")


@functools.partial(
    pl.kernel,
    mesh=_mesh,
    out_type=jax.ShapeDtypeStruct((N_TOTAL, TID_DIM), jnp.float32),
    compiler_params=pltpu.CompilerParams(use_tc_tiling_on_sc=False),
    scratch_types=[
        pltpu.VMEM((CHUNK,), jnp.int32),
        pltpu.VMEM((CHUNK, TID_DIM), jnp.float32),
        pltpu.SemaphoreType.DMA,
    ],
)
def _gather_kernel(table_hbm, idx_hbm, out_hbm, idx_v, rows_v, sem):
    wid = lax.axis_index("s") * 2 + lax.axis_index("c")
    w_base = wid * PER_W

    def body(j, carry):
        base = w_base + j * CHUNK
        pltpu.sync_copy(idx_hbm.at[pl.ds(base, CHUNK)], idx_v)
        pltpu.async_copy(table_hbm.at[idx_v], rows_v, sem).wait()
        pltpu.sync_copy(rows_v, out_hbm.at[pl.ds(base, CHUNK)])
        return carry

    lax.fori_loop(0, NCHUNK, body, 0)


def kernel(tid, node_emb_in, node_emb_out, tid_table, adp_emb):
    idx = tid.reshape(-1).astype(jnp.int32)
    rows = _gather_kernel(tid_table, idx)
    x_tid = rows.reshape(tid.shape + (TID_DIM,))
    return (node_emb_in, node_emb_out, x_tid, adp_emb)
